# baseline (speedup 1.0000x reference)
import jax
import jax.numpy as jnp
from jax.experimental import pallas as pl
from jax.experimental.pallas import tpu as pltpu

_LAMBDA = 1.0
_NEG_INF = float("-inf")
_B = 4096
_D = 1024
_E = 8
_BT = 1024
_NTB = _B // _BT
_NJ = _E + 1


def _moe_kernel(x_ref, wg_ref, bg_ref, w_ref, be_ref, out_ref, cv_ref,
                p_scratch, xbf_scratch, y0_scratch, y1_scratch):
    tb = pl.program_id(0)
    j = pl.program_id(1)

    @pl.when(j == 0)
    def _gate_blk():
        xv = x_ref[...]
        logits = jax.lax.dot_general(
            xv, wg_ref[...], (((1,), (1,)), ((), ())),
            preferred_element_type=jnp.float32) + bg_ref[...]
        idx8 = jax.lax.broadcasted_iota(jnp.int32, (_BT, _E), 1)
        m1 = jnp.max(logits, axis=1, keepdims=True)
        i1 = jnp.min(jnp.where(logits == m1, idx8, _E), axis=1, keepdims=True)
        sel1 = idx8 == i1
        masked = jnp.where(sel1, _NEG_INF, logits)
        m2 = jnp.max(masked, axis=1, keepdims=True)
        i2 = jnp.min(jnp.where(masked == m2, idx8, _E), axis=1, keepdims=True)
        sel2 = idx8 == i2
        e2v = jnp.exp(m2 - m1)
        z = 1.0 + e2v
        p1 = 1.0 / z
        p2 = e2v / z
        pfull = jnp.where(sel1, p1, jnp.where(sel2, p2, 0.0))
        p_scratch[...] = pfull
        xbf_scratch[...] = xv.astype(jnp.bfloat16)

        blk_load = jnp.sum(pfull, axis=0, keepdims=True)

        @pl.when(tb == 0)
        def _():
            cv_ref[0:1, 0:8] = blk_load

        @pl.when(tb != 0)
        def _():
            cv_ref[0:1, 0:8] += blk_load

        @pl.when(tb == _NTB - 1)
        def _():
            load = cv_ref[0:1, 0:8]
            mean = jnp.sum(load) / float(_E)
            var = jnp.sum((load - mean) ** 2) / float(_E - 1)
            cv_ref[...] = jnp.full((8, 128), jnp.sqrt(var) / mean,
                                   jnp.float32)

    @pl.when(j < _E)
    def _mm():
        pe = jnp.sum(
            jnp.where(
                jax.lax.broadcasted_iota(jnp.int32, (_BT, _E), 1) == j,
                p_scratch[...], 0.0),
            axis=1, keepdims=True)
        xs = xbf_scratch[...] * pe.astype(jnp.bfloat16)
        wb = w_ref[0].astype(jnp.bfloat16)
        y = jax.lax.dot_general(
            xs, wb, (((1,), (1,)), ((), ())),
            preferred_element_type=jnp.float32).astype(jnp.bfloat16)

        @pl.when(j % 2 == 0)
        def _():
            y0_scratch[...] = y

        @pl.when(j % 2 == 1)
        def _():
            y1_scratch[...] = y

    @pl.when(j == 1)
    def _init():
        out_ref[...] = y0_scratch[...].astype(jnp.float32)

    @pl.when((j > 1) & (j < _E) & (j % 2 == 0))
    def _acc_odd():
        out_ref[...] += y1_scratch[...].astype(jnp.float32)

    @pl.when((j > 1) & (j < _E) & (j % 2 == 1))
    def _acc_even():
        out_ref[...] += y0_scratch[...].astype(jnp.float32)

    @pl.when(j == _E)
    def _drain():
        pb = jax.lax.dot_general(
            p_scratch[...], be_ref[...], (((1,), (0,)), ((), ())),
            preferred_element_type=jnp.float32)
        out_ref[...] += y1_scratch[...].astype(jnp.float32) + pb


def kernel(x, W_experts, b_experts, W_gate, b_gate):
    out, cvb = pl.pallas_call(
        _moe_kernel,
        grid=(_NTB, _NJ),
        in_specs=[
            pl.BlockSpec((_BT, _D), lambda tb, j: (tb, 0)),
            pl.BlockSpec((_E, _D), lambda tb, j: (0, 0)),
            pl.BlockSpec((1, _E), lambda tb, j: (0, 0)),
            pl.BlockSpec((1, _D, _D),
                         lambda tb, j: (jnp.minimum(j, _E - 1), 0, 0)),
            pl.BlockSpec((_E, _D), lambda tb, j: (0, 0)),
        ],
        out_specs=[
            pl.BlockSpec((_BT, _D), lambda tb, j: (tb, 0)),
            pl.BlockSpec((8, 128), lambda tb, j: (0, 0)),
        ],
        out_shape=[
            jax.ShapeDtypeStruct((_B, _D), jnp.float32),
            jax.ShapeDtypeStruct((8, 128), jnp.float32),
        ],
        scratch_shapes=[
            pltpu.VMEM((_BT, _E), jnp.float32),
            pltpu.VMEM((_BT, _D), jnp.bfloat16),
            pltpu.VMEM((_BT, _D), jnp.bfloat16),
            pltpu.VMEM((_BT, _D), jnp.bfloat16),
        ],
    )(x, W_gate, b_gate.reshape(1, _E), W_experts, b_experts)
    return (out, _LAMBDA * cvb[0, 0])


# speedup vs baseline: 1.1990x; 1.1990x over previous
import jax
import jax.numpy as jnp
from jax.experimental import pallas as pl
from jax.experimental.pallas import tpu as pltpu

_LAMBDA = 1.0
_NEG_INF = float("-inf")
_B = 4096
_D = 1024
_E = 8
_BT = 2048
_NTB = _B // _BT


def _moe_kernel(x_ref, wg_ref, bg_ref, w_ref, be_ref, out_ref, cv_ref,
                p_scratch, xbf_scratch):
    tb = pl.program_id(0)
    e = pl.program_id(1)

    @pl.when(e == 0)
    def _gate_blk():
        xv = x_ref[...]
        logits = jax.lax.dot_general(
            xv, wg_ref[...], (((1,), (1,)), ((), ())),
            preferred_element_type=jnp.float32) + bg_ref[...]
        idx8 = jax.lax.broadcasted_iota(jnp.int32, (_BT, _E), 1)
        m1 = jnp.max(logits, axis=1, keepdims=True)
        i1 = jnp.min(jnp.where(logits == m1, idx8, _E), axis=1, keepdims=True)
        sel1 = idx8 == i1
        masked = jnp.where(sel1, _NEG_INF, logits)
        m2 = jnp.max(masked, axis=1, keepdims=True)
        i2 = jnp.min(jnp.where(masked == m2, idx8, _E), axis=1, keepdims=True)
        sel2 = idx8 == i2
        e2v = jnp.exp(m2 - m1)
        z = 1.0 + e2v
        p1 = 1.0 / z
        p2 = e2v / z
        pfull = jnp.where(sel1, p1, jnp.where(sel2, p2, 0.0))
        p_scratch[...] = pfull
        xbf_scratch[...] = xv.astype(jnp.bfloat16)

        blk_load = jnp.sum(pfull, axis=0, keepdims=True)

        @pl.when(tb == 0)
        def _():
            cv_ref[0:1, 0:8] = blk_load

        @pl.when(tb != 0)
        def _():
            cv_ref[0:1, 0:8] += blk_load

        @pl.when(tb == _NTB - 1)
        def _():
            load = cv_ref[0:1, 0:8]
            mean = jnp.sum(load) / float(_E)
            var = jnp.sum((load - mean) ** 2) / float(_E - 1)
            cv_ref[...] = jnp.full((8, 128), jnp.sqrt(var) / mean,
                                   jnp.float32)

    pe = jnp.sum(
        jnp.where(
            jax.lax.broadcasted_iota(jnp.int32, (_BT, _E), 1) == e,
            p_scratch[...], 0.0),
        axis=1, keepdims=True)
    xs = xbf_scratch[...] * pe.astype(jnp.bfloat16)
    wb = w_ref[0].astype(jnp.bfloat16)
    y = jax.lax.dot_general(
        xs, wb, (((1,), (1,)), ((), ())),
        preferred_element_type=jnp.float32)

    @pl.when(e == 0)
    def _init():
        out_ref[...] = y

    @pl.when((e != 0) & (e != _E - 1))
    def _acc():
        out_ref[...] += y

    @pl.when(e == _E - 1)
    def _last():
        pb = jax.lax.dot_general(
            p_scratch[...], be_ref[...], (((1,), (0,)), ((), ())),
            preferred_element_type=jnp.float32)
        out_ref[...] += y + pb


def kernel(x, W_experts, b_experts, W_gate, b_gate):
    out, cvb = pl.pallas_call(
        _moe_kernel,
        grid=(_NTB, _E),
        in_specs=[
            pl.BlockSpec((_BT, _D), lambda tb, e: (tb, 0)),
            pl.BlockSpec((_E, _D), lambda tb, e: (0, 0)),
            pl.BlockSpec((1, _E), lambda tb, e: (0, 0)),
            pl.BlockSpec((1, _D, _D), lambda tb, e: (e, 0, 0)),
            pl.BlockSpec((_E, _D), lambda tb, e: (0, 0)),
        ],
        out_specs=[
            pl.BlockSpec((_BT, _D), lambda tb, e: (tb, 0)),
            pl.BlockSpec((8, 128), lambda tb, e: (0, 0)),
        ],
        out_shape=[
            jax.ShapeDtypeStruct((_B, _D), jnp.float32),
            jax.ShapeDtypeStruct((8, 128), jnp.float32),
        ],
        scratch_shapes=[
            pltpu.VMEM((_BT, _E), jnp.float32),
            pltpu.VMEM((_BT, _D), jnp.bfloat16),
        ],
    )(x, W_gate, b_gate.reshape(1, _E), W_experts, b_experts)
    return (out, _LAMBDA * cvb[0, 0])
